# baseline (speedup 1.0000x reference)
import jax
import jax.numpy as jnp
from jax import lax
from jax.experimental import pallas as pl
from jax.experimental.pallas import tpu as pltpu
from jax.experimental.pallas import tpu_sc as plsc

_F, _V, _D = 26, 1000, 64
_B, _T = 1024, 50
_NBLK = _T * _F
_NW = 32
_MAXBLK = 41
_QB = 256
_NQ = _B // _QB
_SUB = 128
_ROWS = _T * _F * _D


def _body(idx_hbm, table_hbm, out_hbm, idx_all, gbuf, tbuf, isem, gsem, ssem):
    w = lax.axis_index("s") * 2 + lax.axis_index("c")
    nblk = jnp.where(w < 20, _MAXBLK, _MAXBLK - 1)

    @pl.loop(0, nblk)
    def _stage(k):
        pltpu.async_copy(idx_hbm.at[w + _NW * k], idx_all.at[k], isem)

    @pl.loop(0, nblk)
    def _stage_wait(k):
        pltpu.make_async_copy(idx_hbm.at[w], idx_all.at[0], isem).wait()

    def _gathers(g, q):
        for j in range(_QB // _SUB):
            pltpu.async_copy(
                table_hbm.at[idx_all.at[g, pl.ds(q * _QB + j * _SUB, _SUB)]],
                gbuf.at[q % 2, pl.ds(j * _SUB, _SUB)],
                gsem,
            )

    def _gathers_wait(g, q):
        for j in range(_QB // _SUB):
            pltpu.make_async_copy(
                table_hbm.at[idx_all.at[g, pl.ds(q * _QB + j * _SUB, _SUB)]],
                gbuf.at[q % 2, pl.ds(j * _SUB, _SUB)],
                gsem,
            ).wait()

    def _write_wait():
        pltpu.make_async_copy(
            tbuf.at[0], out_hbm.at[pl.ds(0, _D), pl.ds(0, _QB)], ssem
        ).wait()

    @pl.loop(0, _MAXBLK)
    def _block(g):
        @pl.when(g < nblk)
        def _do():
            m = w + _NW * g
            _gathers(g, 0)
            _gathers(g, 1)
            for q in range(_NQ):
                p = q % 2
                _gathers_wait(g, q)

                @pl.when((g > 0) | (q >= 2))
                def _():
                    _write_wait()

                bidx = [
                    jax.lax.iota(jnp.int32, 16) + 16 * bg
                    for bg in range(_QB // 16)
                ]

                @pl.loop(0, _D)
                def _td(d):
                    didx = jnp.zeros((16,), jnp.int32) + d
                    for bg in range(_QB // 16):
                        vec = plsc.load_gather(gbuf.at[p], [bidx[bg], didx])
                        tbuf[p, d, pl.ds(bg * 16, 16)] = vec

                pltpu.async_copy(
                    tbuf.at[p],
                    out_hbm.at[pl.ds(_D * m, _D), pl.ds(q * _QB, _QB)],
                    ssem,
                )
                if q + 2 < _NQ:
                    _gathers(g, q + 2)

    _write_wait()
    _write_wait()


_gather_t = pl.kernel(
    _body,
    out_type=jax.ShapeDtypeStruct((_ROWS, _B), jnp.float32),
    mesh=plsc.VectorSubcoreMesh(core_axis_name="c", subcore_axis_name="s"),
    scratch_types=[
        pltpu.VMEM((_MAXBLK, _B), jnp.int32),
        pltpu.VMEM((2, _QB, _D), jnp.float32),
        pltpu.VMEM((2, _D, _QB), jnp.float32),
        pltpu.SemaphoreType.DMA,
        pltpu.SemaphoreType.DMA,
        pltpu.SemaphoreType.DMA,
    ],
    compiler_params=pltpu.CompilerParams(
        use_tc_tiling_on_sc=False, needs_layout_passes=False
    ),
)


def kernel(tokens, tables):
    idxp = tokens.transpose(1, 2, 0) + jnp.arange(_F, dtype=jnp.int32)[None, :, None] * _V
    idx = idxp.reshape(_NBLK, _B)
    tab = tables.reshape(_F * _V, _D)
    out = _gather_t(idx, tab)
    return out.reshape(_T, _F, _D, _B).transpose(3, 0, 1, 2)


# speedup vs baseline: 2.1755x; 2.1755x over previous
import jax
import jax.numpy as jnp
from jax import lax
from jax.experimental import pallas as pl
from jax.experimental.pallas import tpu as pltpu
from jax.experimental.pallas import tpu_sc as plsc

_F, _V, _D = 26, 1000, 64
_B, _T = 1024, 50
_NBLK = _T * _F
_NW = 32
_MAXBLK = 41
_QB = 256
_NQ = _B // _QB
_SUB = 128
_ROWS = _T * _F * _D


def _body(idx_hbm, table_hbm, out_hbm, idx_all, gbuf, tbuf, isem, gsem, ssem):
    w = lax.axis_index("s") * 2 + lax.axis_index("c")
    nblk = jnp.where(w < 20, _MAXBLK, _MAXBLK - 1)

    @pl.loop(0, nblk)
    def _stage(k):
        pltpu.async_copy(idx_hbm.at[w + _NW * k], idx_all.at[k], isem)

    @pl.loop(0, nblk)
    def _stage_wait(k):
        pltpu.make_async_copy(idx_hbm.at[w], idx_all.at[0], isem).wait()

    def _gathers(g, q):
        for j in range(_QB // _SUB):
            pltpu.async_copy(
                table_hbm.at[idx_all.at[g, pl.ds(q * _QB + j * _SUB, _SUB)]],
                gbuf.at[q % 2, pl.ds(j * _SUB, _SUB)],
                gsem,
            )

    def _gathers_wait(g, q):
        for j in range(_QB // _SUB):
            pltpu.make_async_copy(
                table_hbm.at[idx_all.at[g, pl.ds(q * _QB + j * _SUB, _SUB)]],
                gbuf.at[q % 2, pl.ds(j * _SUB, _SUB)],
                gsem,
            ).wait()

    def _write_wait():
        pltpu.make_async_copy(
            tbuf.at[0, :, pl.ds(0, _QB)],
            out_hbm.at[pl.ds(0, _D), pl.ds(0, _QB)],
            ssem,
        ).wait()

    @pl.loop(0, _MAXBLK)
    def _block(g):
        @pl.when(g < nblk)
        def _do():
            m = w + _NW * g
            _gathers(g, 0)
            _gathers(g, 1)
            for q in range(_NQ):
                p = q % 2
                _gathers_wait(g, q)

                @pl.when((g > 0) | (q >= 2))
                def _():
                    _write_wait()

                didx = [
                    jax.lax.iota(jnp.int32, 16) + 16 * dg
                    for dg in range(_D // 16)
                ]

                @pl.loop(0, _QB)
                def _tb(b):
                    bsplat = jnp.zeros((16,), jnp.int32) + b
                    for dg in range(_D // 16):
                        vec = gbuf[p, b, pl.ds(dg * 16, 16)]
                        plsc.store_scatter(
                            tbuf.at[p], [didx[dg], bsplat], vec
                        )

                pltpu.async_copy(
                    tbuf.at[p, :, pl.ds(0, _QB)],
                    out_hbm.at[pl.ds(_D * m, _D), pl.ds(q * _QB, _QB)],
                    ssem,
                )
                if q + 2 < _NQ:
                    _gathers(g, q + 2)

    _write_wait()
    _write_wait()


_gather_t = pl.kernel(
    _body,
    out_type=jax.ShapeDtypeStruct((_ROWS, _B), jnp.float32),
    mesh=plsc.VectorSubcoreMesh(core_axis_name="c", subcore_axis_name="s"),
    scratch_types=[
        pltpu.VMEM((_MAXBLK, _B), jnp.int32),
        pltpu.VMEM((2, _QB, _D), jnp.float32),
        pltpu.VMEM((2, _D, _QB + 1), jnp.float32),
        pltpu.SemaphoreType.DMA,
        pltpu.SemaphoreType.DMA,
        pltpu.SemaphoreType.DMA,
    ],
    compiler_params=pltpu.CompilerParams(
        use_tc_tiling_on_sc=False, needs_layout_passes=False
    ),
)


def kernel(tokens, tables):
    idxp = tokens.transpose(1, 2, 0) + jnp.arange(_F, dtype=jnp.int32)[None, :, None] * _V
    idx = idxp.reshape(_NBLK, _B)
    tab = tables.reshape(_F * _V, _D)
    out = _gather_t(idx, tab)
    return out.reshape(_T, _F, _D, _B).transpose(3, 0, 1, 2)


# speedup vs baseline: 2.2597x; 1.0387x over previous
import jax
import jax.numpy as jnp
from jax import lax
from jax.experimental import pallas as pl
from jax.experimental.pallas import tpu as pltpu
from jax.experimental.pallas import tpu_sc as plsc

_F, _V, _D = 26, 1000, 64
_B, _T = 1024, 50
_NBLK = _T * _F
_NW = 32
_MAXBLK = 41
_QB = 256
_NQ = _B // _QB
_SUB = 128
_ROWS = _T * _F * _D


def _body(idx_hbm, table_hbm, out_hbm, idx_all, gbuf, tbuf, isem, gsem, ssem):
    w = lax.axis_index("s") * 2 + lax.axis_index("c")
    nblk = jnp.where(w < 20, _MAXBLK, _MAXBLK - 1)

    @pl.loop(0, nblk)
    def _stage(k):
        pltpu.async_copy(idx_hbm.at[w + _NW * k], idx_all.at[k], isem)

    @pl.loop(0, nblk)
    def _stage_wait(k):
        pltpu.make_async_copy(idx_hbm.at[w], idx_all.at[0], isem).wait()

    def _gathers(g, q):
        for j in range(_QB // _SUB):
            pltpu.async_copy(
                table_hbm.at[idx_all.at[g, pl.ds(q * _QB + j * _SUB, _SUB)]],
                gbuf.at[q % 2, pl.ds(j * _SUB, _SUB)],
                gsem,
            )

    def _gathers_wait(g, q):
        for j in range(_QB // _SUB):
            pltpu.make_async_copy(
                table_hbm.at[idx_all.at[g, pl.ds(q * _QB + j * _SUB, _SUB)]],
                gbuf.at[q % 2, pl.ds(j * _SUB, _SUB)],
                gsem,
            ).wait()

    def _write_wait():
        pltpu.make_async_copy(
            tbuf.at[0, :, pl.ds(0, _QB)],
            out_hbm.at[pl.ds(0, _D), pl.ds(0, _QB)],
            ssem,
        ).wait()

    @pl.loop(0, _MAXBLK)
    def _block(g):
        @pl.when(g < nblk)
        def _do():
            m = w + _NW * g
            _gathers(g, 0)
            _gathers(g, 1)
            for q in range(_NQ):
                p = q % 2
                _gathers_wait(g, q)

                @pl.when((g > 0) | (q >= 2))
                def _():
                    _write_wait()

                didx = [
                    jax.lax.iota(jnp.int32, 16) + 16 * dg
                    for dg in range(_D // 16)
                ]

                @pl.loop(0, _QB // 8)
                def _tb(b8):
                    b0 = b8 * 8
                    bsplat0 = jnp.zeros((16,), jnp.int32) + b0
                    for bj in range(8):
                        bsplat = bsplat0 + bj
                        for dg in range(_D // 16):
                            vec = gbuf[p, b0 + bj, pl.ds(dg * 16, 16)]
                            plsc.store_scatter(
                                tbuf.at[p], [didx[dg], bsplat], vec
                            )

                pltpu.async_copy(
                    tbuf.at[p, :, pl.ds(0, _QB)],
                    out_hbm.at[pl.ds(_D * m, _D), pl.ds(q * _QB, _QB)],
                    ssem,
                )
                if q + 2 < _NQ:
                    _gathers(g, q + 2)

    _write_wait()
    _write_wait()


_gather_t = pl.kernel(
    _body,
    out_type=jax.ShapeDtypeStruct((_ROWS, _B), jnp.float32),
    mesh=plsc.VectorSubcoreMesh(core_axis_name="c", subcore_axis_name="s"),
    scratch_types=[
        pltpu.VMEM((_MAXBLK, _B), jnp.int32),
        pltpu.VMEM((2, _QB, _D), jnp.float32),
        pltpu.VMEM((2, _D, _QB + 1), jnp.float32),
        pltpu.SemaphoreType.DMA,
        pltpu.SemaphoreType.DMA,
        pltpu.SemaphoreType.DMA,
    ],
    compiler_params=pltpu.CompilerParams(
        use_tc_tiling_on_sc=False, needs_layout_passes=False
    ),
)


def kernel(tokens, tables):
    idxp = tokens.transpose(1, 2, 0) + jnp.arange(_F, dtype=jnp.int32)[None, :, None] * _V
    idx = idxp.reshape(_NBLK, _B)
    tab = tables.reshape(_F * _V, _D)
    out = _gather_t(idx, tab)
    return out.reshape(_T, _F, _D, _B).transpose(3, 0, 1, 2)
